# baseline (speedup 1.0000x reference)
import functools

import jax
import jax.numpy as jnp
from jax import lax
from jax.experimental import pallas as pl
from jax.experimental.pallas import tpu as pltpu
from jax.experimental.pallas import tpu_sc as plsc

N_NODES = 10000
N_EDGES = 320000
D = 128
HALF = D // 2
NUM_TYPES = 4

NC = 2
NS = 16
NW = NC * NS
LANES = 16

P_PER_S = HALF // NS
E_PER_C = N_EDGES // NC
CHUNK = 8000
N_CHUNKS = E_PER_C // CHUNK
GROUPS = CHUNK // LANES


def _pack_pairs(d_f32):
    lo = lax.bitcast_convert_type(d_f32[:, :HALF].astype(jnp.bfloat16),
                                  jnp.uint16).astype(jnp.uint32)
    hi = lax.bitcast_convert_type(d_f32[:, HALF:].astype(jnp.bfloat16),
                                  jnp.uint16).astype(jnp.uint32)
    return lax.bitcast_convert_type(lo | (hi << 16), jnp.int32).T


def _project_body(x_ref, wu_ref, bu_ref, wv_ref, bv_ref, s_ref, r_ref, e_ref,
                  upk_ref, vpk_ref, pidx_ref):
    xb = x_ref[...]
    du = jnp.dot(xb, wu_ref[...], preferred_element_type=jnp.float32) + bu_ref[...]
    dv = jnp.dot(xb, wv_ref[...], preferred_element_type=jnp.float32) + bv_ref[...]
    upk_ref[...] = _pack_pairs(du)
    vpk_ref[...] = _pack_pairs(dv)
    pidx_ref[...] = s_ref[...] | (r_ref[...] << 14) | (e_ref[...] << 28)


def _project(x, wu, bu2, wv, bv2, s2d, r2d, e2d):
    whole = lambda shape: pl.BlockSpec(shape, lambda: tuple(0 for _ in shape))
    return pl.pallas_call(
        _project_body,
        grid=(),
        in_specs=[
            whole((N_NODES, D)),
            whole((D, D)),
            whole((1, D)),
            whole((D, D)),
            whole((1, D)),
            whole((N_EDGES // D, D)),
            whole((N_EDGES // D, D)),
            whole((N_EDGES // D, D)),
        ],
        out_specs=[
            whole((HALF, N_NODES)),
            whole((HALF, N_NODES)),
            whole((N_EDGES // D, D)),
        ],
        out_shape=[
            jax.ShapeDtypeStruct((HALF, N_NODES), jnp.int32),
            jax.ShapeDtypeStruct((HALF, N_NODES), jnp.int32),
            jax.ShapeDtypeStruct((N_EDGES // D, D), jnp.int32),
        ],
    )(x, wu, bu2, wv, bv2, s2d, r2d, e2d)


def _reduce_body(part_ref, out_ref):
    out_ref[...] = jnp.sum(part_ref[...].astype(jnp.float32), axis=0,
                           keepdims=True)


def _reduce(partials_bf16):
    B = 12800
    grid = (N_EDGES // B,)
    return pl.pallas_call(
        _reduce_body,
        grid=grid,
        in_specs=[pl.BlockSpec((NS, B), lambda i: (0, i))],
        out_specs=pl.BlockSpec((1, B), lambda i: (0, i)),
        out_shape=jax.ShapeDtypeStruct((1, N_EDGES), jnp.float32),
    )(partials_bf16)


ESTRIDE = 33


def _edge_body(upk_hbm, vpk_hbm, emb_hbm, pidx_hbm, part_hbm,
               utbl, vtbl, etbl, erep, sh_idx,
               i0, p0, i1, p1, sem0, sem1, osem):
    cid = lax.axis_index("c")
    sid = lax.axis_index("s")
    lanes = lax.iota(jnp.int32, LANES)
    lanebase = lanes * ESTRIDE

    tbase = pl.multiple_of(sid * (P_PER_S * N_NODES), P_PER_S * N_NODES)
    pltpu.sync_copy(upk_hbm.at[pl.ds(tbase, P_PER_S * N_NODES)], utbl)
    pltpu.sync_copy(vpk_hbm.at[pl.ds(tbase, P_PER_S * N_NODES)], vtbl)
    pltpu.sync_copy(emb_hbm, etbl)

    @pl.when(sid == 0)
    def _():
        hbase = pl.multiple_of(cid * E_PER_C, E_PER_C)
        pltpu.sync_copy(pidx_hbm.at[pl.ds(hbase, E_PER_C)], sh_idx)

    for c in range(NUM_TYPES * P_PER_S):
        t, dl = c // P_PER_S, c % P_PER_S
        dglob = sid * P_PER_S + dl
        lo = plsc.load_gather(etbl, [jnp.full((LANES,), t * D, jnp.int32) + dglob])
        hi = plsc.load_gather(etbl, [jnp.full((LANES,), t * D + HALF, jnp.int32) + dglob])
        pk = plsc.bitcast(plsc.pack(lo, hi, format=plsc.PackFormat.INTERLEAVED),
                          jnp.int32)
        plsc.store_scatter(erep, [lanebase + c], pk)

    plsc.subcore_barrier()

    def loads(k, ib, sem):
        off = pl.multiple_of(k * CHUNK, CHUNK)
        pltpu.async_copy(sh_idx.at[pl.ds(off, CHUNK)], ib, sem)

    def drain(ib, sem):
        pltpu.make_async_copy(sh_idx.at[pl.ds(0, CHUNK)], ib, sem).wait()

    def compute(ib, pb):
        def one_group(pk):
            sv = lax.bitwise_and(pk, 0x3FFF)
            rv = lax.bitwise_and(lax.shift_right_logical(pk, 14), 0x3FFF)
            evb = lanebase + lax.shift_left(lax.shift_right_logical(pk, 28), 2)

            acc = jnp.zeros((LANES,), jnp.float32)
            for dl in range(P_PER_S):
                u = plsc.load_gather(utbl, [sv + dl * N_NODES])
                v = plsc.load_gather(vtbl, [rv + dl * N_NODES])
                prod = plsc.bitcast(u, jnp.bfloat16) * plsc.bitcast(v, jnp.bfloat16)
                plo, phi = plsc.unpack(prod, format=plsc.PackFormat.INTERLEAVED)
                epk = plsc.load_gather(erep, [evb + dl])
                elo, ehi = plsc.unpack(plsc.bitcast(epk, jnp.bfloat16),
                                       format=plsc.PackFormat.INTERLEAVED)
                acc = acc + plo * elo + phi * ehi
            return acc

        def pair_body(gp, carry2):
            acc_a = one_group(ib[pl.ds(gp * (2 * LANES), LANES)])
            acc_b = one_group(ib[pl.ds(gp * (2 * LANES) + LANES, LANES)])
            pb[pl.ds(gp * LANES, LANES)] = plsc.bitcast(
                plsc.pack(acc_a, acc_b, format=plsc.PackFormat.INTERLEAVED),
                jnp.int32)
            return carry2

        lax.fori_loop(0, GROUPS // 2, pair_body, 0)

    def store(k, pb):
        off = pl.multiple_of(
            (sid * N_EDGES + cid * E_PER_C + k * CHUNK) // 2, CHUNK // 2)
        pltpu.async_copy(pb, part_hbm.at[pl.ds(off, CHUNK // 2)], osem)

    def store_wait(pb):
        pltpu.make_async_copy(pb, part_hbm.at[pl.ds(0, CHUNK // 2)], osem).wait()

    loads(0, i0, sem0)

    def chunk_pair(p, carry):
        k = p * 2
        loads(k + 1, i1, sem1)
        drain(i0, sem0)

        @pl.when(p > 0)
        def _():
            store_wait(p0)

        compute(i0, p0)
        store(k, p0)

        @pl.when(k + 2 < N_CHUNKS)
        def _():
            loads(k + 2, i0, sem0)

        drain(i1, sem1)

        @pl.when(p > 0)
        def _():
            store_wait(p1)

        compute(i1, p1)
        store(k + 1, p1)
        return carry

    lax.fori_loop(0, N_CHUNKS // 2, chunk_pair, 0)
    store_wait(p0)
    store_wait(p1)


@functools.cache
def _edge_kernel():
    return pl.kernel(
        _edge_body,
        out_type=jax.ShapeDtypeStruct((NS * N_EDGES // 2,), jnp.int32),
        mesh=plsc.VectorSubcoreMesh(core_axis_name="c", subcore_axis_name="s"),
        scratch_types=[
            pltpu.VMEM((P_PER_S * N_NODES,), jnp.int32),
            pltpu.VMEM((P_PER_S * N_NODES,), jnp.int32),
            pltpu.VMEM((NUM_TYPES * D,), jnp.float32),
            pltpu.VMEM((LANES * ESTRIDE,), jnp.int32),
            pltpu.VMEM_SHARED((E_PER_C,), jnp.int32),
            pltpu.VMEM((CHUNK,), jnp.int32),
            pltpu.VMEM((CHUNK // 2,), jnp.int32),
            pltpu.VMEM((CHUNK,), jnp.int32),
            pltpu.VMEM((CHUNK // 2,), jnp.int32),
            pltpu.SemaphoreType.DMA,
            pltpu.SemaphoreType.DMA,
            pltpu.SemaphoreType.DMA,
        ],
        compiler_params=pltpu.CompilerParams(needs_layout_passes=False),
    )


def kernel(x, senders, receivers, edge_feature, emb, Wu, bu, Wv, bv):
    upk, vpk, pidx = _project(
        x, Wu, bu.reshape(1, D), Wv, bv.reshape(1, D),
        senders.reshape(N_EDGES // D, D),
        receivers.reshape(N_EDGES // D, D),
        edge_feature.reshape(N_EDGES // D, D),
    )
    partials = _edge_kernel()(
        upk.reshape(HALF * N_NODES), vpk.reshape(HALF * N_NODES),
        emb.reshape(NUM_TYPES * D), pidx.reshape(N_EDGES))
    part_bf = lax.bitcast_convert_type(
        partials.reshape(NS, N_EDGES // 2), jnp.bfloat16).reshape(NS, N_EDGES)
    permuted = _reduce(part_bf).reshape(N_EDGES)
    return permuted.reshape(N_EDGES // 32, LANES, 2).transpose(0, 2, 1).reshape(N_EDGES)


# speedup vs baseline: 2.9760x; 2.9760x over previous
import functools

import jax
import jax.numpy as jnp
from jax import lax
from jax.experimental import pallas as pl
from jax.experimental.pallas import tpu as pltpu
from jax.experimental.pallas import tpu_sc as plsc

N_NODES = 10000
N_EDGES = 320000
D = 128
HALF = D // 2
NUM_TYPES = 4

NC = 2
NS = 16
NW = NC * NS
LANES = 16

P_PER_S = HALF // NS
E_PER_C = N_EDGES // NC
CHUNK = 8000
N_CHUNKS = E_PER_C // CHUNK
GROUPS = CHUNK // LANES


def _pack_pairs(d_f32):
    lo = lax.bitcast_convert_type(d_f32[:, :HALF].astype(jnp.bfloat16),
                                  jnp.uint16).astype(jnp.uint32)
    hi = lax.bitcast_convert_type(d_f32[:, HALF:].astype(jnp.bfloat16),
                                  jnp.uint16).astype(jnp.uint32)
    return lax.bitcast_convert_type(lo | (hi << 16), jnp.int32).T


def _project_body(x_ref, wu_ref, bu_ref, wv_ref, bv_ref, s_ref, r_ref, e_ref,
                  upk_ref, vpk_ref, pidx_ref):
    xb = x_ref[...]
    du = jnp.dot(xb, wu_ref[...], preferred_element_type=jnp.float32) + bu_ref[...]
    dv = jnp.dot(xb, wv_ref[...], preferred_element_type=jnp.float32) + bv_ref[...]
    upk_ref[...] = _pack_pairs(du)
    vpk_ref[...] = _pack_pairs(dv)
    pidx_ref[...] = s_ref[...] | (r_ref[...] << 14) | (e_ref[...] << 28)


def _project(x, wu, bu2, wv, bv2, s2d, r2d, e2d):
    whole = lambda shape: pl.BlockSpec(shape, lambda: tuple(0 for _ in shape))
    return pl.pallas_call(
        _project_body,
        grid=(),
        in_specs=[
            whole((N_NODES, D)),
            whole((D, D)),
            whole((1, D)),
            whole((D, D)),
            whole((1, D)),
            whole((N_EDGES // D, D)),
            whole((N_EDGES // D, D)),
            whole((N_EDGES // D, D)),
        ],
        out_specs=[
            whole((HALF, N_NODES)),
            whole((HALF, N_NODES)),
            whole((N_EDGES // D, D)),
        ],
        out_shape=[
            jax.ShapeDtypeStruct((HALF, N_NODES), jnp.int32),
            jax.ShapeDtypeStruct((HALF, N_NODES), jnp.int32),
            jax.ShapeDtypeStruct((N_EDGES // D, D), jnp.int32),
        ],
    )(x, wu, bu2, wv, bv2, s2d, r2d, e2d)


def _reduce_body(part_ref, out_ref):
    out_ref[...] = jnp.sum(part_ref[...], axis=0, keepdims=True)


def _reduce(partials):
    B = 12800
    grid = (N_EDGES // B,)
    return pl.pallas_call(
        _reduce_body,
        grid=grid,
        in_specs=[pl.BlockSpec((NS, B), lambda i: (0, i))],
        out_specs=pl.BlockSpec((1, B), lambda i: (0, i)),
        out_shape=jax.ShapeDtypeStruct((1, N_EDGES), jnp.float32),
    )(partials)


ESTRIDE = 33


def _edge_body(upk_hbm, vpk_hbm, emb_hbm, pidx_hbm, part_hbm,
               utbl, vtbl, etbl, erep, sh_idx,
               i0, p0, i1, p1, sem0, sem1, osem):
    cid = lax.axis_index("c")
    sid = lax.axis_index("s")
    lanes = lax.iota(jnp.int32, LANES)
    lanebase = lanes * ESTRIDE

    tbase = pl.multiple_of(sid * (P_PER_S * N_NODES), P_PER_S * N_NODES)
    pltpu.sync_copy(upk_hbm.at[pl.ds(tbase, P_PER_S * N_NODES)], utbl)
    pltpu.sync_copy(vpk_hbm.at[pl.ds(tbase, P_PER_S * N_NODES)], vtbl)
    pltpu.sync_copy(emb_hbm, etbl)

    @pl.when(sid == 0)
    def _():
        hbase = pl.multiple_of(cid * E_PER_C, E_PER_C)
        pltpu.sync_copy(pidx_hbm.at[pl.ds(hbase, E_PER_C)], sh_idx)

    for c in range(NUM_TYPES * 8):
        t, pos = c // 8, c % 8
        dglob = sid * P_PER_S + (pos % P_PER_S) + (HALF if pos >= P_PER_S else 0)
        val = plsc.load_gather(etbl, [jnp.full((LANES,), t * D, jnp.int32) + dglob])
        plsc.store_scatter(erep, [lanebase + c], val)

    plsc.subcore_barrier()

    def loads(k, ib, sem):
        off = pl.multiple_of(k * CHUNK, CHUNK)
        pltpu.async_copy(sh_idx.at[pl.ds(off, CHUNK)], ib, sem)

    def drain(ib, sem):
        pltpu.make_async_copy(sh_idx.at[pl.ds(0, CHUNK)], ib, sem).wait()

    def compute(ib, pb):
        def group_body(g, carry2):
            sl = pl.ds(g * LANES, LANES)
            pk = ib[sl]
            sv = lax.bitwise_and(pk, 0x3FFF)
            rv = lax.bitwise_and(lax.shift_right_logical(pk, 14), 0x3FFF)
            evb = lanebase + lax.shift_left(lax.shift_right_logical(pk, 28), 3)

            acc = jnp.zeros((LANES,), jnp.float32)
            for dl in range(P_PER_S):
                u = plsc.load_gather(utbl, [sv + dl * N_NODES])
                v = plsc.load_gather(vtbl, [rv + dl * N_NODES])
                prod = plsc.bitcast(u, jnp.bfloat16) * plsc.bitcast(v, jnp.bfloat16)
                plo, phi = plsc.unpack(prod, format=plsc.PackFormat.INTERLEAVED)
                elo = plsc.load_gather(erep, [evb + dl])
                ehi = plsc.load_gather(erep, [evb + (P_PER_S + dl)])
                acc = acc + plo * elo + phi * ehi
            pb[sl] = acc
            return carry2

        lax.fori_loop(0, GROUPS, group_body, 0)

    def store(k, pb):
        off = pl.multiple_of(sid * N_EDGES + cid * E_PER_C + k * CHUNK, CHUNK)
        pltpu.async_copy(pb, part_hbm.at[pl.ds(off, CHUNK)], osem)

    def store_wait(pb):
        pltpu.make_async_copy(pb, part_hbm.at[pl.ds(0, CHUNK)], osem).wait()

    loads(0, i0, sem0)

    def chunk_pair(p, carry):
        k = p * 2
        loads(k + 1, i1, sem1)
        drain(i0, sem0)

        @pl.when(p > 0)
        def _():
            store_wait(p0)

        compute(i0, p0)
        store(k, p0)

        @pl.when(k + 2 < N_CHUNKS)
        def _():
            loads(k + 2, i0, sem0)

        drain(i1, sem1)

        @pl.when(p > 0)
        def _():
            store_wait(p1)

        compute(i1, p1)
        store(k + 1, p1)
        return carry

    lax.fori_loop(0, N_CHUNKS // 2, chunk_pair, 0)
    store_wait(p0)
    store_wait(p1)


@functools.cache
def _edge_kernel():
    return pl.kernel(
        _edge_body,
        out_type=jax.ShapeDtypeStruct((NS * N_EDGES,), jnp.float32),
        mesh=plsc.VectorSubcoreMesh(core_axis_name="c", subcore_axis_name="s"),
        scratch_types=[
            pltpu.VMEM((P_PER_S * N_NODES,), jnp.int32),
            pltpu.VMEM((P_PER_S * N_NODES,), jnp.int32),
            pltpu.VMEM((NUM_TYPES * D,), jnp.float32),
            pltpu.VMEM((LANES * ESTRIDE,), jnp.float32),
            pltpu.VMEM_SHARED((E_PER_C,), jnp.int32),
            pltpu.VMEM((CHUNK,), jnp.int32),
            pltpu.VMEM((CHUNK,), jnp.float32),
            pltpu.VMEM((CHUNK,), jnp.int32),
            pltpu.VMEM((CHUNK,), jnp.float32),
            pltpu.SemaphoreType.DMA,
            pltpu.SemaphoreType.DMA,
            pltpu.SemaphoreType.DMA,
        ],
        compiler_params=pltpu.CompilerParams(needs_layout_passes=False),
    )


def kernel(x, senders, receivers, edge_feature, emb, Wu, bu, Wv, bv):
    upk, vpk, pidx = _project(
        x, Wu, bu.reshape(1, D), Wv, bv.reshape(1, D),
        senders.reshape(N_EDGES // D, D),
        receivers.reshape(N_EDGES // D, D),
        edge_feature.reshape(N_EDGES // D, D),
    )
    partials = _edge_kernel()(
        upk.reshape(HALF * N_NODES), vpk.reshape(HALF * N_NODES),
        emb.reshape(NUM_TYPES * D), pidx.reshape(N_EDGES))
    return _reduce(partials.reshape(NS, N_EDGES)).reshape(N_EDGES)


# speedup vs baseline: 3.0391x; 1.0212x over previous
import functools

import jax
import jax.numpy as jnp
from jax import lax
from jax.experimental import pallas as pl
from jax.experimental.pallas import tpu as pltpu
from jax.experimental.pallas import tpu_sc as plsc

N_NODES = 10000
N_EDGES = 320000
D = 128
HALF = D // 2
NUM_TYPES = 4

NC = 2
NS = 16
NW = NC * NS
LANES = 16

P_PER_S = HALF // NS
E_PER_C = N_EDGES // NC
CHUNK = 8000
N_CHUNKS = E_PER_C // CHUNK
GROUPS = CHUNK // LANES


def _pack_pairs(d_f32):
    lo = lax.bitcast_convert_type(d_f32[:, :HALF].astype(jnp.bfloat16),
                                  jnp.uint16).astype(jnp.uint32)
    hi = lax.bitcast_convert_type(d_f32[:, HALF:].astype(jnp.bfloat16),
                                  jnp.uint16).astype(jnp.uint32)
    return lax.bitcast_convert_type(lo | (hi << 16), jnp.int32).T


def _project_body(x_ref, wu_ref, bu_ref, wv_ref, bv_ref, s_ref, r_ref, e_ref,
                  upk_ref, vpk_ref, pidx_ref):
    xb = x_ref[...]
    du = jnp.dot(xb, wu_ref[...], preferred_element_type=jnp.float32) + bu_ref[...]
    dv = jnp.dot(xb, wv_ref[...], preferred_element_type=jnp.float32) + bv_ref[...]
    upk_ref[...] = _pack_pairs(du)
    vpk_ref[...] = _pack_pairs(dv)
    pidx_ref[...] = s_ref[...] | (r_ref[...] << 14) | (e_ref[...] << 28)


def _project(x, wu, bu2, wv, bv2, s2d, r2d, e2d):
    whole = lambda shape: pl.BlockSpec(shape, lambda: tuple(0 for _ in shape))
    return pl.pallas_call(
        _project_body,
        grid=(),
        in_specs=[
            whole((N_NODES, D)),
            whole((D, D)),
            whole((1, D)),
            whole((D, D)),
            whole((1, D)),
            whole((N_EDGES // D, D)),
            whole((N_EDGES // D, D)),
            whole((N_EDGES // D, D)),
        ],
        out_specs=[
            whole((HALF, N_NODES)),
            whole((HALF, N_NODES)),
            whole((N_EDGES // D, D)),
        ],
        out_shape=[
            jax.ShapeDtypeStruct((HALF, N_NODES), jnp.int32),
            jax.ShapeDtypeStruct((HALF, N_NODES), jnp.int32),
            jax.ShapeDtypeStruct((N_EDGES // D, D), jnp.int32),
        ],
    )(x, wu, bu2, wv, bv2, s2d, r2d, e2d)


def _reduce_body(part_ref, out_ref):
    out_ref[...] = jnp.sum(part_ref[...], axis=0, keepdims=True)


def _reduce(partials):
    B = 12800
    grid = (N_EDGES // B,)
    return pl.pallas_call(
        _reduce_body,
        grid=grid,
        in_specs=[pl.BlockSpec((NS, B), lambda i: (0, i))],
        out_specs=pl.BlockSpec((1, B), lambda i: (0, i)),
        out_shape=jax.ShapeDtypeStruct((1, N_EDGES), jnp.float32),
    )(partials)


ESTRIDE = 33


def _edge_body(upk_hbm, vpk_hbm, emb_hbm, pidx_hbm, part_hbm,
               utbl, vtbl, etbl, erep, sh_idx,
               i0, p0, i1, p1, sem0, sem1, osem):
    cid = lax.axis_index("c")
    sid = lax.axis_index("s")
    lanes = lax.iota(jnp.int32, LANES)
    lanebase = lanes * ESTRIDE

    tbase = pl.multiple_of(sid * (P_PER_S * N_NODES), P_PER_S * N_NODES)
    pltpu.sync_copy(upk_hbm.at[pl.ds(tbase, P_PER_S * N_NODES)], utbl)
    pltpu.sync_copy(vpk_hbm.at[pl.ds(tbase, P_PER_S * N_NODES)], vtbl)
    pltpu.sync_copy(emb_hbm, etbl)

    @pl.when(sid == 0)
    def _():
        hbase = pl.multiple_of(cid * E_PER_C, E_PER_C)
        pltpu.sync_copy(pidx_hbm.at[pl.ds(hbase, E_PER_C)], sh_idx)

    for c in range(NUM_TYPES * P_PER_S):
        t, dl = c // P_PER_S, c % P_PER_S
        dglob = sid * P_PER_S + dl
        lo = plsc.load_gather(etbl, [jnp.full((LANES,), t * D, jnp.int32) + dglob])
        hi = plsc.load_gather(etbl, [jnp.full((LANES,), t * D + HALF, jnp.int32) + dglob])
        pk = plsc.bitcast(plsc.pack(lo, hi, format=plsc.PackFormat.INTERLEAVED),
                          jnp.int32)
        plsc.store_scatter(erep, [lanebase + c], pk)

    plsc.subcore_barrier()

    def loads(k, ib, sem):
        off = pl.multiple_of(k * CHUNK, CHUNK)
        pltpu.async_copy(sh_idx.at[pl.ds(off, CHUNK)], ib, sem)

    def drain(ib, sem):
        pltpu.make_async_copy(sh_idx.at[pl.ds(0, CHUNK)], ib, sem).wait()

    def compute(ib, pb):
        def group_body(g, carry2):
            sl = pl.ds(g * LANES, LANES)
            pk = ib[sl]
            sv = lax.bitwise_and(pk, 0x3FFF)
            rv = lax.bitwise_and(lax.shift_right_logical(pk, 14), 0x3FFF)
            evb = lanebase + lax.shift_left(lax.shift_right_logical(pk, 28), 2)

            acc = jnp.zeros((LANES,), jnp.float32)
            for dl in range(P_PER_S):
                u = plsc.load_gather(utbl, [sv + dl * N_NODES])
                v = plsc.load_gather(vtbl, [rv + dl * N_NODES])
                prod = plsc.bitcast(u, jnp.bfloat16) * plsc.bitcast(v, jnp.bfloat16)
                plo, phi = plsc.unpack(prod, format=plsc.PackFormat.INTERLEAVED)
                epk = plsc.load_gather(erep, [evb + dl])
                elo, ehi = plsc.unpack(plsc.bitcast(epk, jnp.bfloat16),
                                       format=plsc.PackFormat.INTERLEAVED)
                acc = acc + plo * elo + phi * ehi
            pb[sl] = acc
            return carry2

        lax.fori_loop(0, GROUPS, group_body, 0)

    def store(k, pb):
        off = pl.multiple_of(sid * N_EDGES + cid * E_PER_C + k * CHUNK, CHUNK)
        pltpu.async_copy(pb, part_hbm.at[pl.ds(off, CHUNK)], osem)

    def store_wait(pb):
        pltpu.make_async_copy(pb, part_hbm.at[pl.ds(0, CHUNK)], osem).wait()

    loads(0, i0, sem0)

    def chunk_pair(p, carry):
        k = p * 2
        loads(k + 1, i1, sem1)
        drain(i0, sem0)

        @pl.when(p > 0)
        def _():
            store_wait(p0)

        compute(i0, p0)
        store(k, p0)

        @pl.when(k + 2 < N_CHUNKS)
        def _():
            loads(k + 2, i0, sem0)

        drain(i1, sem1)

        @pl.when(p > 0)
        def _():
            store_wait(p1)

        compute(i1, p1)
        store(k + 1, p1)
        return carry

    lax.fori_loop(0, N_CHUNKS // 2, chunk_pair, 0)
    store_wait(p0)
    store_wait(p1)


@functools.cache
def _edge_kernel():
    return pl.kernel(
        _edge_body,
        out_type=jax.ShapeDtypeStruct((NS * N_EDGES,), jnp.float32),
        mesh=plsc.VectorSubcoreMesh(core_axis_name="c", subcore_axis_name="s"),
        scratch_types=[
            pltpu.VMEM((P_PER_S * N_NODES,), jnp.int32),
            pltpu.VMEM((P_PER_S * N_NODES,), jnp.int32),
            pltpu.VMEM((NUM_TYPES * D,), jnp.float32),
            pltpu.VMEM((LANES * ESTRIDE,), jnp.int32),
            pltpu.VMEM_SHARED((E_PER_C,), jnp.int32),
            pltpu.VMEM((CHUNK,), jnp.int32),
            pltpu.VMEM((CHUNK,), jnp.float32),
            pltpu.VMEM((CHUNK,), jnp.int32),
            pltpu.VMEM((CHUNK,), jnp.float32),
            pltpu.SemaphoreType.DMA,
            pltpu.SemaphoreType.DMA,
            pltpu.SemaphoreType.DMA,
        ],
        compiler_params=pltpu.CompilerParams(needs_layout_passes=False),
    )


def kernel(x, senders, receivers, edge_feature, emb, Wu, bu, Wv, bv):
    upk, vpk, pidx = _project(
        x, Wu, bu.reshape(1, D), Wv, bv.reshape(1, D),
        senders.reshape(N_EDGES // D, D),
        receivers.reshape(N_EDGES // D, D),
        edge_feature.reshape(N_EDGES // D, D),
    )
    partials = _edge_kernel()(
        upk.reshape(HALF * N_NODES), vpk.reshape(HALF * N_NODES),
        emb.reshape(NUM_TYPES * D), pidx.reshape(N_EDGES))
    return _reduce(partials.reshape(NS, N_EDGES)).reshape(N_EDGES)
